# baseline (speedup 1.0000x reference)
import functools

import jax
import jax.numpy as jnp
from jax import lax
from jax.experimental import pallas as pl
from jax.experimental.pallas import tpu as pltpu
from jax.experimental.pallas import tpu_sc as plsc

B = 16384
RT, RS, RP = 8, 12, 4
NC, NS, L = 2, 16, 16
NW = NC * NS
BW = B // NW
CHUNK = 128
NCH = BW // CHUNK
GPC = CHUNK // L
PD = 128
NIDX = 8


def _sc_body(stk, corev, et, ex, ey, eu, out,
             idx_v, pad_t, pad_x, pad_y, pad_u,
             core_v, out_v, sem):
    wid = lax.axis_index("s") * NC + lax.axis_index("c")
    base = wid * BW

    pltpu.sync_copy(stk.at[wid], idx_v)
    pltpu.sync_copy(corev, core_v)

    lane = lax.iota(jnp.int32, L)

    cvecs = [core_v[pl.ds(v * L, L)] for v in range(RT * RS * RP // L)]

    def cscal(n):
        return cvecs[n // L][n % L]

    def run_chunk(c):
        s = pl.ds(c * CHUNK, CHUNK)
        descs = [
            pltpu.async_copy(et.at[idx_v.at[0].at[s]], pad_t, sem),
            pltpu.async_copy(ex.at[idx_v.at[1].at[s]], pad_x, sem),
            pltpu.async_copy(ey.at[idx_v.at[2].at[s]], pad_y, sem),
            pltpu.async_copy(eu.at[idx_v.at[3].at[s]], pad_u, sem),
        ]
        for d in descs:
            d.wait()

        def group(g, carry):
            row = lane + g * L
            gsl = pl.ds(c * CHUNK + g * L, L)
            to = idx_v[4, gsl]
            xo = idx_v[5, gsl]
            yo = idx_v[6, gsl]
            uo = idx_v[7, gsl]
            tcols = [plsc.load_gather(pad_t, [row, to + i]) for i in range(RT)]
            xcols = [plsc.load_gather(pad_x, [row, xo + j]) for j in range(RS)]
            ycols = [plsc.load_gather(pad_y, [row, yo + j]) for j in range(RS)]
            qcols = [plsc.load_gather(pad_u, [row, uo + k]) for k in range(RP)]
            sxy = [xcols[j] * ycols[j] for j in range(RS)]
            acc = None
            for i in range(RT):
                e_i = None
                for j in range(RS):
                    off = (i * RS + j) * RP
                    m = cscal(off) * qcols[0]
                    for k in range(1, RP):
                        m = m + cscal(off + k) * qcols[k]
                    term = m * sxy[j]
                    e_i = term if e_i is None else e_i + term
                contrib = tcols[i] * e_i
                acc = contrib if acc is None else acc + contrib
            out_v[gsl] = acc
            return carry

        lax.fori_loop(0, GPC, group, 0)

    for c in range(NCH):
        run_chunk(c)

    pltpu.sync_copy(out_v, out.at[pl.ds(base, BW)])


def kernel(indices, core_tensor, edge_time, edge_space_x, edge_space_y,
           edge_physics):
    tu = indices[:, 0]
    xu = indices[:, 1]
    yu = indices[:, 2]
    uu = indices[:, 3]
    stk = jnp.stack([
        tu >> 4, xu // 10, yu // 10, uu >> 5,
        (tu & 15) << 3, (xu % 10) * 12, (yu % 10) * 12, (uu & 31) << 2,
    ]).reshape(NIDX, NW, BW).transpose(1, 0, 2)
    corev = core_tensor.reshape(RT * RS * RP)
    etp = edge_time.reshape(100000 // 16, PD)
    exp_ = jnp.pad(edge_space_x.reshape(10000, 120), ((0, 0), (0, 8)))
    eyp = jnp.pad(edge_space_y.reshape(10000, 120), ((0, 0), (0, 8)))
    eup = edge_physics.reshape(100000 // 32, PD)

    mesh = plsc.VectorSubcoreMesh(core_axis_name="c", subcore_axis_name="s")
    call = functools.partial(
        pl.kernel,
        mesh=mesh,
        compiler_params=pltpu.CompilerParams(
            needs_layout_passes=False, use_tc_tiling_on_sc=True),
        out_type=jax.ShapeDtypeStruct((B,), jnp.float32),
        scratch_types=[
            pltpu.VMEM((NIDX, BW), jnp.int32),
            pltpu.VMEM((CHUNK, PD), jnp.float32),
            pltpu.VMEM((CHUNK, PD), jnp.float32),
            pltpu.VMEM((CHUNK, PD), jnp.float32),
            pltpu.VMEM((CHUNK, PD), jnp.float32),
            pltpu.VMEM((RT * RS * RP,), jnp.float32),
            pltpu.VMEM((BW,), jnp.float32),
            pltpu.SemaphoreType.DMA,
        ],
    )(_sc_body)
    return call(stk, corev, etp, exp_, eyp, eup)


# speedup vs baseline: 1.6401x; 1.6401x over previous
import functools

import jax
import jax.numpy as jnp
from jax import lax
from jax.experimental import pallas as pl
from jax.experimental.pallas import tpu as pltpu
from jax.experimental.pallas import tpu_sc as plsc

B = 16384
V = 100000
RT, RS, RP = 8, 12, 4
NC, NS, L = 2, 16, 16
NW = NC * NS
BW = B // NW
CHUNK = 128
NCH = BW // CHUNK
GPC = CHUNK // L
PD = 128
NIDX = 8
def _repack_body(tt, xt, yt, ut, ot, ox, oy, ou):
    def one(in_ref, out_ref, w, wp, pack):
        r = V // pack
        ipad = jnp.eye(w, wp, dtype=jnp.float32)
        for m in range(pack):
            band = lax.dot_general(
                in_ref[:, m * r:(m + 1) * r], ipad,
                ((( 0,), (0,)), ((), ())),
                preferred_element_type=jnp.float32)
            out_ref[:, m * wp:(m + 1) * wp] = band

    one(tt, ot, RT, 8, 16)
    one(xt, ox, RS, 16, 8)
    one(yt, oy, RS, 16, 8)
    one(ut, ou, RP, 4, 32)


def _repack(tt, xt, yt, ut):
    return pl.pallas_call(
        _repack_body,
        out_shape=[
            jax.ShapeDtypeStruct((V // 16, PD), jnp.float32),
            jax.ShapeDtypeStruct((V // 8, PD), jnp.float32),
            jax.ShapeDtypeStruct((V // 8, PD), jnp.float32),
            jax.ShapeDtypeStruct((V // 32, PD), jnp.float32),
        ],
    )(tt, xt, yt, ut)


def _sc_body(stk, corev, et, ex, ey, eu, out,
             idx_v, pad_t, pad_x, pad_y, pad_u,
             core_v, out_v, sem):
    wid = lax.axis_index("s") * NC + lax.axis_index("c")
    base = wid * BW

    pltpu.sync_copy(stk.at[wid], idx_v)
    pltpu.sync_copy(corev, core_v)

    lane = lax.iota(jnp.int32, L)

    cvecs = [core_v[pl.ds(v * L, L)] for v in range(RT * RS * RP // L)]

    def cscal(n):
        return cvecs[n // L][n % L]

    def run_chunk(c):
        s = pl.ds(c * CHUNK, CHUNK)
        descs = [
            pltpu.async_copy(et.at[idx_v.at[0].at[s]], pad_t, sem),
            pltpu.async_copy(ex.at[idx_v.at[1].at[s]], pad_x, sem),
            pltpu.async_copy(ey.at[idx_v.at[2].at[s]], pad_y, sem),
            pltpu.async_copy(eu.at[idx_v.at[3].at[s]], pad_u, sem),
        ]
        for d in descs:
            d.wait()

        def group(g, carry):
            row = lane + g * L
            gsl = pl.ds(c * CHUNK + g * L, L)
            to = idx_v[4, gsl]
            xo = idx_v[5, gsl]
            yo = idx_v[6, gsl]
            uo = idx_v[7, gsl]
            tcols = [plsc.load_gather(pad_t, [row, to + i]) for i in range(RT)]
            xcols = [plsc.load_gather(pad_x, [row, xo + j]) for j in range(RS)]
            ycols = [plsc.load_gather(pad_y, [row, yo + j]) for j in range(RS)]
            qcols = [plsc.load_gather(pad_u, [row, uo + k]) for k in range(RP)]
            sxy = [xcols[j] * ycols[j] for j in range(RS)]
            acc = None
            for i in range(RT):
                e_i = None
                for j in range(RS):
                    off = (i * RS + j) * RP
                    m = cscal(off) * qcols[0]
                    for k in range(1, RP):
                        m = m + cscal(off + k) * qcols[k]
                    term = m * sxy[j]
                    e_i = term if e_i is None else e_i + term
                contrib = tcols[i] * e_i
                acc = contrib if acc is None else acc + contrib
            out_v[gsl] = acc
            return carry

        lax.fori_loop(0, GPC, group, 0)

    for c in range(NCH):
        run_chunk(c)

    pltpu.sync_copy(out_v, out.at[pl.ds(base, BW)])


def kernel(indices, core_tensor, edge_time, edge_space_x, edge_space_y,
           edge_physics):
    tu = indices[:, 0]
    xu = indices[:, 1]
    yu = indices[:, 2]
    uu = indices[:, 3]
    rt_, rx_, rp_ = V // 16, V // 8, V // 32
    stk = jnp.stack([
        tu % rt_, xu % rx_, yu % rx_, uu % rp_,
        (tu // rt_) * 8, (xu // rx_) * 16, (yu // rx_) * 16, (uu // rp_) * 4,
    ]).reshape(NIDX, NW, BW).transpose(1, 0, 2)
    corev = core_tensor.reshape(RT * RS * RP)

    etp, exp_, eyp, eup = _repack(
        edge_time.T, edge_space_x.T, edge_space_y.T, edge_physics.T)

    mesh = plsc.VectorSubcoreMesh(core_axis_name="c", subcore_axis_name="s")
    call = functools.partial(
        pl.kernel,
        mesh=mesh,
        compiler_params=pltpu.CompilerParams(
            needs_layout_passes=False, use_tc_tiling_on_sc=True),
        out_type=jax.ShapeDtypeStruct((B,), jnp.float32),
        scratch_types=[
            pltpu.VMEM((NIDX, BW), jnp.int32),
            pltpu.VMEM((CHUNK, PD), jnp.float32),
            pltpu.VMEM((CHUNK, PD), jnp.float32),
            pltpu.VMEM((CHUNK, PD), jnp.float32),
            pltpu.VMEM((CHUNK, PD), jnp.float32),
            pltpu.VMEM((RT * RS * RP,), jnp.float32),
            pltpu.VMEM((BW,), jnp.float32),
            pltpu.SemaphoreType.DMA,
        ],
    )(_sc_body)
    return call(stk, corev, etp, exp_, eyp, eup)


# speedup vs baseline: 3.7310x; 2.2749x over previous
import functools

import numpy as np

import jax
import jax.numpy as jnp
from jax import lax
from jax.experimental import pallas as pl
from jax.experimental.pallas import tpu as pltpu
from jax.experimental.pallas import tpu_sc as plsc

B = 16384
V = 100000
RT, RS, RP = 8, 12, 4
NC, NS, L = 2, 16, 16
NW = NC * NS
BW = B // NW
CHUNK = 128
NCH = BW // CHUNK
GPC = CHUNK // L
PD = 128
NIDX = 8
def _place(w, wp, pack):
    p = np.zeros((w * pack, PD), np.float32)
    for m in range(pack):
        for i in range(w):
            p[m * w + i, m * wp + i] = 1.0
    return p


def _repack_body(tt, xt, yt, ut, pt_, px_, py_, pu_, ot, ox, oy, ou):
    def one(in_ref, place_ref, out_ref, pack):
        r = V // pack
        lhs = jnp.concatenate(
            [in_ref[:, m * r:(m + 1) * r] for m in range(pack)], axis=0)
        out_ref[...] = lax.dot_general(
            lhs, place_ref[...],
            (((0,), (0,)), ((), ())),
            preferred_element_type=jnp.float32)

    one(tt, pt_, ot, 16)
    one(xt, px_, ox, 8)
    one(yt, py_, oy, 8)
    one(ut, pu_, ou, 32)


def _repack(tt, xt, yt, ut):
    return pl.pallas_call(
        _repack_body,
        out_shape=[
            jax.ShapeDtypeStruct((V // 16, PD), jnp.float32),
            jax.ShapeDtypeStruct((V // 8, PD), jnp.float32),
            jax.ShapeDtypeStruct((V // 8, PD), jnp.float32),
            jax.ShapeDtypeStruct((V // 32, PD), jnp.float32),
        ],
    )(tt, xt, yt, ut,
      jnp.asarray(_place(RT, 8, 16)), jnp.asarray(_place(RS, 16, 8)),
      jnp.asarray(_place(RS, 16, 8)), jnp.asarray(_place(RP, 4, 32)))


def _sc_body(stk, corev, et, ex, ey, eu, out,
             idx_v, pad_t, pad_x, pad_y, pad_u,
             core_v, out_v, sem):
    wid = lax.axis_index("s") * NC + lax.axis_index("c")
    base = wid * BW

    pltpu.sync_copy(stk.at[wid], idx_v)
    pltpu.sync_copy(corev, core_v)

    lane = lax.iota(jnp.int32, L)

    cvecs = [core_v[pl.ds(v * L, L)] for v in range(RT * RS * RP // L)]

    def cscal(n):
        return cvecs[n // L][n % L]

    def run_chunk(c):
        s = pl.ds(c * CHUNK, CHUNK)
        descs = [
            pltpu.async_copy(et.at[idx_v.at[0].at[s]], pad_t, sem),
            pltpu.async_copy(ex.at[idx_v.at[1].at[s]], pad_x, sem),
            pltpu.async_copy(ey.at[idx_v.at[2].at[s]], pad_y, sem),
            pltpu.async_copy(eu.at[idx_v.at[3].at[s]], pad_u, sem),
        ]
        for d in descs:
            d.wait()

        def group(g, carry):
            row = lane + g * L
            gsl = pl.ds(c * CHUNK + g * L, L)
            to = idx_v[4, gsl]
            xo = idx_v[5, gsl]
            yo = idx_v[6, gsl]
            uo = idx_v[7, gsl]
            tcols = [plsc.load_gather(pad_t, [row, to + i]) for i in range(RT)]
            xcols = [plsc.load_gather(pad_x, [row, xo + j]) for j in range(RS)]
            ycols = [plsc.load_gather(pad_y, [row, yo + j]) for j in range(RS)]
            qcols = [plsc.load_gather(pad_u, [row, uo + k]) for k in range(RP)]
            sxy = [xcols[j] * ycols[j] for j in range(RS)]
            acc = None
            for i in range(RT):
                e_i = None
                for j in range(RS):
                    off = (i * RS + j) * RP
                    m = cscal(off) * qcols[0]
                    for k in range(1, RP):
                        m = m + cscal(off + k) * qcols[k]
                    term = m * sxy[j]
                    e_i = term if e_i is None else e_i + term
                contrib = tcols[i] * e_i
                acc = contrib if acc is None else acc + contrib
            out_v[gsl] = acc
            return carry

        lax.fori_loop(0, GPC, group, 0)

    for c in range(NCH):
        run_chunk(c)

    pltpu.sync_copy(out_v, out.at[pl.ds(base, BW)])


def kernel(indices, core_tensor, edge_time, edge_space_x, edge_space_y,
           edge_physics):
    tu = indices[:, 0]
    xu = indices[:, 1]
    yu = indices[:, 2]
    uu = indices[:, 3]
    rt_, rx_, rp_ = V // 16, V // 8, V // 32
    stk = jnp.stack([
        tu % rt_, xu % rx_, yu % rx_, uu % rp_,
        (tu // rt_) * 8, (xu // rx_) * 16, (yu // rx_) * 16, (uu // rp_) * 4,
    ]).reshape(NIDX, NW, BW).transpose(1, 0, 2)
    corev = core_tensor.reshape(RT * RS * RP)

    etp, exp_, eyp, eup = _repack(
        edge_time.T, edge_space_x.T, edge_space_y.T, edge_physics.T)

    mesh = plsc.VectorSubcoreMesh(core_axis_name="c", subcore_axis_name="s")
    call = functools.partial(
        pl.kernel,
        mesh=mesh,
        compiler_params=pltpu.CompilerParams(
            needs_layout_passes=False, use_tc_tiling_on_sc=True),
        out_type=jax.ShapeDtypeStruct((B,), jnp.float32),
        scratch_types=[
            pltpu.VMEM((NIDX, BW), jnp.int32),
            pltpu.VMEM((CHUNK, PD), jnp.float32),
            pltpu.VMEM((CHUNK, PD), jnp.float32),
            pltpu.VMEM((CHUNK, PD), jnp.float32),
            pltpu.VMEM((CHUNK, PD), jnp.float32),
            pltpu.VMEM((RT * RS * RP,), jnp.float32),
            pltpu.VMEM((BW,), jnp.float32),
            pltpu.SemaphoreType.DMA,
        ],
    )(_sc_body)
    return call(stk, corev, etp, exp_, eyp, eup)


# speedup vs baseline: 4.0682x; 1.0904x over previous
import functools

import numpy as np

import jax
import jax.numpy as jnp
from jax import lax
from jax.experimental import pallas as pl
from jax.experimental.pallas import tpu as pltpu
from jax.experimental.pallas import tpu_sc as plsc

B = 16384
V = 100000
RT, RS, RP = 8, 12, 4
NC, NS, L = 2, 16, 16
NW = NC * NS
BW = B // NW
CHUNK = 64
NCH = BW // CHUNK
GPC = CHUNK // L
PD = 128
NIDX = 8
def _place(w, wp, pack):
    p = np.zeros((w * pack, PD), np.float32)
    for m in range(pack):
        for i in range(w):
            p[m * w + i, m * wp + i] = 1.0
    return p


def _repack_body(tt, xt, yt, ut, pt_, px_, py_, pu_, ot, ox, oy, ou):
    def one(in_ref, place_ref, out_ref, pack):
        r = V // pack
        lhs = jnp.concatenate(
            [in_ref[:, m * r:(m + 1) * r] for m in range(pack)], axis=0)
        out_ref[...] = lax.dot_general(
            lhs, place_ref[...],
            (((0,), (0,)), ((), ())),
            preferred_element_type=jnp.float32)

    one(tt, pt_, ot, 16)
    one(xt, px_, ox, 8)
    one(yt, py_, oy, 8)
    one(ut, pu_, ou, 32)


def _repack(tt, xt, yt, ut):
    return pl.pallas_call(
        _repack_body,
        out_shape=[
            jax.ShapeDtypeStruct((V // 16, PD), jnp.float32),
            jax.ShapeDtypeStruct((V // 8, PD), jnp.float32),
            jax.ShapeDtypeStruct((V // 8, PD), jnp.float32),
            jax.ShapeDtypeStruct((V // 32, PD), jnp.float32),
        ],
    )(tt, xt, yt, ut,
      jnp.asarray(_place(RT, 8, 16)), jnp.asarray(_place(RS, 16, 8)),
      jnp.asarray(_place(RS, 16, 8)), jnp.asarray(_place(RP, 4, 32)))


def _sc_body(stk, corev, et, ex, ey, eu, out,
             idx_v,
             pt0, px0, py0, pu0, pt1, px1, py1, pu1,
             core_v, out_v, sem0, sem1):
    wid = lax.axis_index("s") * NC + lax.axis_index("c")
    base = wid * BW

    pltpu.sync_copy(stk.at[wid], idx_v)
    pltpu.sync_copy(corev, core_v)

    lane = lax.iota(jnp.int32, L)

    cvecs = [core_v[pl.ds(v * L, L)] for v in range(RT * RS * RP // L)]

    def cscal(n):
        return cvecs[n // L][n % L]

    bufs = [(pt0, px0, py0, pu0), (pt1, px1, py1, pu1)]
    sems = [sem0, sem1]

    def fire(c):
        s = pl.ds(c * CHUNK, CHUNK)
        b, sm = bufs[c % 2], sems[c % 2]
        return [
            pltpu.async_copy(et.at[idx_v.at[0].at[s]], b[0], sm),
            pltpu.async_copy(ex.at[idx_v.at[1].at[s]], b[1], sm),
            pltpu.async_copy(ey.at[idx_v.at[2].at[s]], b[2], sm),
            pltpu.async_copy(eu.at[idx_v.at[3].at[s]], b[3], sm),
        ]

    def compute(c):
        pad_t, pad_x, pad_y, pad_u = bufs[c % 2]

        def group(g, carry):
            row = lane + g * L
            gsl = pl.ds(c * CHUNK + g * L, L)
            to = idx_v[4, gsl]
            xo = idx_v[5, gsl]
            yo = idx_v[6, gsl]
            uo = idx_v[7, gsl]
            tcols = [plsc.load_gather(pad_t, [row, to + i]) for i in range(RT)]
            xcols = [plsc.load_gather(pad_x, [row, xo + j]) for j in range(RS)]
            ycols = [plsc.load_gather(pad_y, [row, yo + j]) for j in range(RS)]
            qcols = [plsc.load_gather(pad_u, [row, uo + k]) for k in range(RP)]
            sxy = [xcols[j] * ycols[j] for j in range(RS)]
            acc = None
            for i in range(RT):
                e_i = None
                for j in range(RS):
                    off = (i * RS + j) * RP
                    m = cscal(off) * qcols[0]
                    for k in range(1, RP):
                        m = m + cscal(off + k) * qcols[k]
                    term = m * sxy[j]
                    e_i = term if e_i is None else e_i + term
                contrib = tcols[i] * e_i
                acc = contrib if acc is None else acc + contrib
            out_v[gsl] = acc
            return carry

        lax.fori_loop(0, GPC, group, 0)

    descs = {0: fire(0)}
    for c in range(NCH):
        if c + 1 < NCH:
            descs[c + 1] = fire(c + 1)
        for d in descs.pop(c):
            d.wait()
        compute(c)

    pltpu.sync_copy(out_v, out.at[pl.ds(base, BW)])


def kernel(indices, core_tensor, edge_time, edge_space_x, edge_space_y,
           edge_physics):
    tu = indices[:, 0]
    xu = indices[:, 1]
    yu = indices[:, 2]
    uu = indices[:, 3]
    rt_, rx_, rp_ = V // 16, V // 8, V // 32
    stk = jnp.stack([
        tu % rt_, xu % rx_, yu % rx_, uu % rp_,
        (tu // rt_) * 8, (xu // rx_) * 16, (yu // rx_) * 16, (uu // rp_) * 4,
    ]).reshape(NIDX, NW, BW).transpose(1, 0, 2)
    corev = core_tensor.reshape(RT * RS * RP)

    etp, exp_, eyp, eup = _repack(
        edge_time.T, edge_space_x.T, edge_space_y.T, edge_physics.T)

    mesh = plsc.VectorSubcoreMesh(core_axis_name="c", subcore_axis_name="s")
    call = functools.partial(
        pl.kernel,
        mesh=mesh,
        compiler_params=pltpu.CompilerParams(
            needs_layout_passes=False, use_tc_tiling_on_sc=True),
        out_type=jax.ShapeDtypeStruct((B,), jnp.float32),
        scratch_types=[
            pltpu.VMEM((NIDX, BW), jnp.int32),
            pltpu.VMEM((CHUNK, PD), jnp.float32),
            pltpu.VMEM((CHUNK, PD), jnp.float32),
            pltpu.VMEM((CHUNK, PD), jnp.float32),
            pltpu.VMEM((CHUNK, PD), jnp.float32),
            pltpu.VMEM((CHUNK, PD), jnp.float32),
            pltpu.VMEM((CHUNK, PD), jnp.float32),
            pltpu.VMEM((CHUNK, PD), jnp.float32),
            pltpu.VMEM((CHUNK, PD), jnp.float32),
            pltpu.VMEM((RT * RS * RP,), jnp.float32),
            pltpu.VMEM((BW,), jnp.float32),
            pltpu.SemaphoreType.DMA,
            pltpu.SemaphoreType.DMA,
        ],
    )(_sc_body)
    return call(stk, corev, etp, exp_, eyp, eup)
